# baseline (speedup 1.0000x reference)
import functools

import jax
import jax.numpy as jnp
from jax import lax
from jax.experimental import pallas as pl
from jax.experimental.pallas import tpu as pltpu

_GRAPHS_PER_STEP = 16
_DIAG_MULT = 1.0100501670841680


def _mha_body(x_ref, adj_ref, wkqv_ref, wo_ref, o_ref, *, n_heads, key_dim):
    H, K = n_heads, key_dim
    G, N, D = x_ref.shape
    hk = H * K

    row = lax.broadcasted_iota(jnp.int32, (N, N), 0)
    col = lax.broadcasted_iota(jnp.int32, (N, N), 1)
    eye_b = row == col

    QB = 128

    ones_nk = jnp.ones((N, K), jnp.bfloat16)

    x = x_ref[...].reshape(G * N, D).astype(jnp.bfloat16)
    kqv_all = jnp.dot(x, wkqv_ref[...],
                      preferred_element_type=jnp.float32).astype(jnp.bfloat16)

    for g in range(G):
        kqv = kqv_all[g * N:(g + 1) * N, :]
        mult = jnp.where(eye_b, _DIAG_MULT, adj_ref[g])
        pv_rows = []
        for qb in range(N // QB):
            q0 = qb * QB
            mult_blk = mult[q0:q0 + QB, :]
            pv_parts = []
            for h in range(H):
                k_h = kqv[:, h * K:(h + 1) * K]
                q_b = kqv[q0:q0 + QB, hk + h * K:hk + (h + 1) * K]
                v_h = kqv[:, 2 * hk + h * K:2 * hk + (h + 1) * K]
                s = lax.dot_general(q_b, k_h, (((1,), (1,)), ((), ())),
                                    preferred_element_type=jnp.float32)
                em = jnp.exp2(jnp.minimum(s, 100.0)) * mult_blk
                v_ext = jnp.concatenate([v_h, ones_nk], axis=-1)
                pvx = jnp.dot(em.astype(jnp.bfloat16), v_ext,
                              preferred_element_type=jnp.float32)
                r = pl.reciprocal(pvx[:, K:], approx=True)
                pv_parts.append(pvx[:, :K] * r)
            pv_rows.append(jnp.concatenate(pv_parts, axis=-1))
        pv = jnp.concatenate(pv_rows, axis=0).astype(jnp.bfloat16)
        o_ref[g] = jnp.dot(pv, wo_ref[...],
                           preferred_element_type=jnp.float32
                           ).astype(o_ref.dtype)


def kernel(node_matrices, adjacencies, Wk, Wq, Wv, Wo):
    B, N, D = node_matrices.shape
    H, _, K = Wk.shape
    O = Wo.shape[-1]
    G = _GRAPHS_PER_STEP
    assert B % G == 0

    def _pack(w):
        return jnp.transpose(w, (1, 0, 2)).reshape(D, H * K)

    q_scale = 1.4426950408889634 / (K ** 0.5)
    w_kqv = jnp.concatenate([_pack(Wk), _pack(Wq) * q_scale, _pack(Wv)],
                            axis=-1).astype(jnp.bfloat16)
    wo_flat = Wo.reshape(H * K, O).astype(jnp.bfloat16)

    body = functools.partial(_mha_body, n_heads=H, key_dim=K)
    return pl.pallas_call(
        body,
        out_shape=jax.ShapeDtypeStruct((B, N, O), jnp.float32),
        grid=(B // G,),
        in_specs=[
            pl.BlockSpec((G, N, D), lambda b: (b, 0, 0)),
            pl.BlockSpec((G, N, N), lambda b: (b, 0, 0)),
            pl.BlockSpec((D, 3 * H * K), lambda b: (0, 0)),
            pl.BlockSpec((H * K, O), lambda b: (0, 0)),
        ],
        out_specs=pl.BlockSpec((G, N, O), lambda b: (b, 0, 0)),
        compiler_params=pltpu.CompilerParams(
            dimension_semantics=("parallel",)),
    )(node_matrices, adjacencies, w_kqv, wo_flat)


# speedup vs baseline: 1.0510x; 1.0510x over previous
import functools

import jax
import jax.numpy as jnp
from jax import lax
from jax.experimental import pallas as pl
from jax.experimental.pallas import tpu as pltpu

_GRAPHS_PER_STEP = 8
_DIAG_MULT = 1.0100501670841680


def _mha_body(x_ref, adj_ref, wkqv_ref, wo_ref, o_ref, *, n_heads, key_dim):
    H, K = n_heads, key_dim
    G, N, D = x_ref.shape
    hk = H * K

    row = lax.broadcasted_iota(jnp.int32, (N, N), 0)
    col = lax.broadcasted_iota(jnp.int32, (N, N), 1)
    eye_b = row == col

    QB = 128

    ones_nk = jnp.ones((N, K), jnp.bfloat16)

    x = x_ref[...].reshape(G * N, D).astype(jnp.bfloat16)
    kqv_all = jnp.dot(x, wkqv_ref[...],
                      preferred_element_type=jnp.float32).astype(jnp.bfloat16)

    for g in range(G):
        kqv = kqv_all[g * N:(g + 1) * N, :]
        mult = jnp.where(eye_b, _DIAG_MULT, adj_ref[g])
        pv_rows = []
        for qb in range(N // QB):
            q0 = qb * QB
            mult_blk = mult[q0:q0 + QB, :]
            pv_parts = []
            for h in range(H):
                k_h = kqv[:, h * K:(h + 1) * K]
                q_b = kqv[q0:q0 + QB, hk + h * K:hk + (h + 1) * K]
                v_h = kqv[:, 2 * hk + h * K:2 * hk + (h + 1) * K]
                s = lax.dot_general(q_b, k_h, (((1,), (1,)), ((), ())),
                                    preferred_element_type=jnp.float32)
                em = jnp.exp2(jnp.minimum(s, 100.0)) * mult_blk
                v_ext = jnp.concatenate([v_h, ones_nk], axis=-1)
                pvx = jnp.dot(em.astype(jnp.bfloat16), v_ext,
                              preferred_element_type=jnp.float32)
                r = pl.reciprocal(pvx[:, K:], approx=True)
                pv_parts.append(pvx[:, :K] * r)
            pv_rows.append(jnp.concatenate(pv_parts, axis=-1))
        pv = jnp.concatenate(pv_rows, axis=0).astype(jnp.bfloat16)
        o_ref[g] = jnp.dot(pv, wo_ref[...],
                           preferred_element_type=jnp.float32
                           ).astype(o_ref.dtype)


def kernel(node_matrices, adjacencies, Wk, Wq, Wv, Wo):
    B, N, D = node_matrices.shape
    H, _, K = Wk.shape
    O = Wo.shape[-1]
    G = _GRAPHS_PER_STEP
    assert B % G == 0

    def _pack(w):
        return jnp.transpose(w, (1, 0, 2)).reshape(D, H * K)

    q_scale = 1.4426950408889634 / (K ** 0.5)
    w_kqv = jnp.concatenate([_pack(Wk), _pack(Wq) * q_scale, _pack(Wv)],
                            axis=-1).astype(jnp.bfloat16)
    wo_flat = Wo.reshape(H * K, O).astype(jnp.bfloat16)

    body = functools.partial(_mha_body, n_heads=H, key_dim=K)
    return pl.pallas_call(
        body,
        out_shape=jax.ShapeDtypeStruct((B, N, O), jnp.float32),
        grid=(B // G,),
        in_specs=[
            pl.BlockSpec((G, N, D), lambda b: (b, 0, 0)),
            pl.BlockSpec((G, N, N), lambda b: (b, 0, 0)),
            pl.BlockSpec((D, 3 * H * K), lambda b: (0, 0)),
            pl.BlockSpec((H * K, O), lambda b: (0, 0)),
        ],
        out_specs=pl.BlockSpec((G, N, O), lambda b: (b, 0, 0)),
        compiler_params=pltpu.CompilerParams(
            dimension_semantics=("parallel",)),
    )(node_matrices, adjacencies, w_kqv, wo_flat)
